# Initial kernel scaffold; baseline (speedup 1.0000x reference)
import functools

import jax
import jax.numpy as jnp
from jax import lax
from jax.experimental import pallas as pl
from jax.experimental.pallas import tpu as pltpu
from jax.experimental.pallas import tpu_sc as plsc

D = 64
_info = plsc.get_sparse_core_info()
NC, NS = _info.num_cores, _info.num_subcores
NW = NC * NS

G = 128


def _make_sc_call(n_rows: int):
    per_w = n_rows // NW
    n_groups = per_w // G
    mesh = plsc.VectorSubcoreMesh(core_axis_name="c", subcore_axis_name="s")

    @functools.partial(
        pl.kernel,
        out_type=(
            jax.ShapeDtypeStruct((n_rows, D), jnp.float32),
            jax.ShapeDtypeStruct((n_rows, D), jnp.float32),
        ),
        mesh=mesh,
        scratch_types=[
            pltpu.VMEM((G,), jnp.int32),
            pltpu.VMEM((G,), jnp.int32),
            pltpu.VMEM((G, D), jnp.float32),
            pltpu.VMEM((G, D), jnp.float32),
            pltpu.SemaphoreType.DMA,
            pltpu.SemaphoreType.DMA,
        ],
    )
    def sc_call(users_h, types_h, utab_h, ttab_h, out1_h, out2_h,
                idx_u, idx_t, urows, trows, sem_u, sem_t):
        wid = lax.axis_index("s") * NC + lax.axis_index("c")
        base_w = wid * per_w

        def step(g, carry):
            base = base_w + g * G
            pltpu.sync_copy(users_h.at[pl.ds(base, G)], idx_u)
            pltpu.sync_copy(types_h.at[pl.ds(base, G)], idx_t)
            cu = pltpu.async_copy(utab_h.at[idx_u], urows, sem_u)
            ct = pltpu.async_copy(ttab_h.at[idx_t], trows, sem_t)
            cu.wait()
            ct.wait()

            def add_row(i, c2):
                for c in range(D // 16):
                    sl = pl.ds(c * 16, 16)
                    urows[i, sl] = urows[i, sl] + trows[i, sl]
                return c2

            lax.fori_loop(0, G, add_row, 0)
            pltpu.sync_copy(urows, out1_h.at[pl.ds(base, G)])
            pltpu.sync_copy(trows, out2_h.at[pl.ds(base, G)])
            return carry

        lax.fori_loop(0, n_groups, step, 0)

    return sc_call


_sc_call = _make_sc_call(4096 * 50)


@jax.jit
def kernel(users, items, types, user_table, item_table, type_table):
    del items, item_table
    b, l = users.shape
    u = users.reshape(-1).astype(jnp.int32)
    t = types.reshape(-1).astype(jnp.int32)
    out1, out2 = _sc_call(u, t, user_table, type_table)
    return (out1.reshape(b, l, D), out2.reshape(b, l, D))


# speedup vs baseline: 1.7087x; 1.7087x over previous
import functools

import jax
import jax.numpy as jnp
from jax import lax
from jax.experimental import pallas as pl
from jax.experimental.pallas import tpu as pltpu
from jax.experimental.pallas import tpu_sc as plsc

D = 64
_info = plsc.get_sparse_core_info()
NC, NS = _info.num_cores, _info.num_subcores
NW = NC * NS

G = 128


def _make_sc_call(n_rows: int):
    per_w = n_rows // NW
    n_groups = per_w // G
    mesh = plsc.VectorSubcoreMesh(core_axis_name="c", subcore_axis_name="s")

    @functools.partial(
        pl.kernel,
        out_type=(
            jax.ShapeDtypeStruct((n_rows, D), jnp.float32),
            jax.ShapeDtypeStruct((n_rows, D), jnp.float32),
        ),
        mesh=mesh,
        scratch_types=[
            pltpu.VMEM((G,), jnp.int32),
            pltpu.VMEM((G,), jnp.int32),
            pltpu.VMEM((G, D), jnp.float32),
            pltpu.VMEM((G, D), jnp.float32),
            pltpu.SemaphoreType.DMA,
            pltpu.SemaphoreType.DMA,
        ],
        compiler_params=pltpu.CompilerParams(use_tc_tiling_on_sc=False),
    )
    def sc_call(users_h, types_h, utab_h, ttab_h, out1_h, out2_h,
                idx_u, idx_t, urows, trows, sem_u, sem_t):
        wid = lax.axis_index("s") * NC + lax.axis_index("c")
        base_w = wid * per_w

        def step(g, carry):
            base = base_w + g * G
            pltpu.sync_copy(users_h.at[pl.ds(base, G)], idx_u)
            pltpu.sync_copy(types_h.at[pl.ds(base, G)], idx_t)
            cu = pltpu.async_copy(utab_h.at[idx_u], urows, sem_u)
            ct = pltpu.async_copy(ttab_h.at[idx_t], trows, sem_t)
            cu.wait()
            ct.wait()

            def add_row(i, c2):
                for c in range(D // 16):
                    sl = pl.ds(c * 16, 16)
                    urows[i, sl] = urows[i, sl] + trows[i, sl]
                return c2

            lax.fori_loop(0, G, add_row, 0)
            pltpu.sync_copy(urows, out1_h.at[pl.ds(base, G)])
            pltpu.sync_copy(trows, out2_h.at[pl.ds(base, G)])
            return carry

        lax.fori_loop(0, n_groups, step, 0)

    return sc_call


_sc_call = _make_sc_call(4096 * 50)


@jax.jit
def kernel(users, items, types, user_table, item_table, type_table):
    del items, item_table
    b, l = users.shape
    u = users.reshape(-1).astype(jnp.int32)
    t = types.reshape(-1).astype(jnp.int32)
    out1, out2 = _sc_call(u, t, user_table, type_table)
    return (out1.reshape(b, l, D), out2.reshape(b, l, D))


# speedup vs baseline: 1.7458x; 1.0217x over previous
import functools

import jax
import jax.numpy as jnp
from jax import lax
from jax.experimental import pallas as pl
from jax.experimental.pallas import tpu as pltpu
from jax.experimental.pallas import tpu_sc as plsc

D = 64
_info = plsc.get_sparse_core_info()
NC, NS = _info.num_cores, _info.num_subcores
NW = NC * NS

G = 128
NBUF = 5
DIST = 3


def _make_sc_call(n_rows: int):
    per_w = n_rows // NW
    n_groups = per_w // G
    assert n_groups % NBUF == 0
    mesh = plsc.VectorSubcoreMesh(core_axis_name="c", subcore_axis_name="s")

    @functools.partial(
        pl.kernel,
        out_type=(
            jax.ShapeDtypeStruct((n_rows, D), jnp.float32),
            jax.ShapeDtypeStruct((n_rows, D), jnp.float32),
        ),
        mesh=mesh,
        scratch_types=[
            pltpu.VMEM((NBUF, G), jnp.int32),
            pltpu.VMEM((NBUF, G), jnp.int32),
            pltpu.VMEM((NBUF, G, D), jnp.float32),
            pltpu.VMEM((NBUF, G, D), jnp.float32),
            pltpu.SemaphoreType.DMA((NBUF,)),
            pltpu.SemaphoreType.DMA((NBUF,)),
            pltpu.SemaphoreType.DMA((NBUF,)),
            pltpu.SemaphoreType.DMA((NBUF,)),
        ],
        compiler_params=pltpu.CompilerParams(use_tc_tiling_on_sc=False),
    )
    def sc_call(users_h, types_h, utab_h, ttab_h, out1_h, out2_h,
                idx_u, idx_t, urows, trows, gsem_u, gsem_t, osem1, osem2):
        wid = lax.axis_index("s") * NC + lax.axis_index("c")
        base_w = wid * per_w

        def fire(j, b):
            base = base_w + j * G
            pltpu.sync_copy(users_h.at[pl.ds(base, G)], idx_u.at[b])
            pltpu.sync_copy(types_h.at[pl.ds(base, G)], idx_t.at[b])
            pltpu.async_copy(utab_h.at[idx_u.at[b]], urows.at[b], gsem_u.at[b])
            pltpu.async_copy(ttab_h.at[idx_t.at[b]], trows.at[b], gsem_t.at[b])

        for b in range(DIST):
            fire(b, b)

        def super_step(s, carry):
            for b in range(NBUF):
                i = s * NBUF + b
                base = base_w + i * G
                pltpu.make_async_copy(
                    utab_h.at[idx_u.at[b]], urows.at[b], gsem_u.at[b]).wait()
                pltpu.make_async_copy(
                    ttab_h.at[idx_t.at[b]], trows.at[b], gsem_t.at[b]).wait()

                def add_row(r, c2, _b=b):
                    for c in range(D // 16):
                        sl = pl.ds(c * 16, 16)
                        urows[_b, r, sl] = urows[_b, r, sl] + trows[_b, r, sl]
                    return c2

                lax.fori_loop(0, G, add_row, 0)
                pltpu.async_copy(urows.at[b], out1_h.at[pl.ds(base, G)],
                                 osem1.at[b])
                pltpu.async_copy(trows.at[b], out2_h.at[pl.ds(base, G)],
                                 osem2.at[b])

                j = i + DIST
                b2 = (b + DIST) % NBUF

                @pl.when(j < n_groups)
                def _():
                    @pl.when(j >= NBUF)
                    def _():
                        pltpu.make_async_copy(
                            urows.at[b2],
                            out1_h.at[pl.ds(base_w + (j - NBUF) * G, G)],
                            osem1.at[b2]).wait()
                        pltpu.make_async_copy(
                            trows.at[b2],
                            out2_h.at[pl.ds(base_w + (j - NBUF) * G, G)],
                            osem2.at[b2]).wait()
                    fire(j, b2)
            return carry

        lax.fori_loop(0, n_groups // NBUF, super_step, 0)

        for b in range(NBUF):
            g_last = n_groups - NBUF + b
            pltpu.make_async_copy(
                urows.at[b], out1_h.at[pl.ds(base_w + g_last * G, G)],
                osem1.at[b]).wait()
            pltpu.make_async_copy(
                trows.at[b], out2_h.at[pl.ds(base_w + g_last * G, G)],
                osem2.at[b]).wait()

    return sc_call


_sc_call = _make_sc_call(4096 * 50)


@jax.jit
def kernel(users, items, types, user_table, item_table, type_table):
    del items, item_table
    b, l = users.shape
    u = users.reshape(-1).astype(jnp.int32)
    t = types.reshape(-1).astype(jnp.int32)
    out1, out2 = _sc_call(u, t, user_table, type_table)
    return (out1.reshape(b, l, D), out2.reshape(b, l, D))


# speedup vs baseline: 1.7852x; 1.0226x over previous
import functools

import jax
import jax.numpy as jnp
from jax import lax
from jax.experimental import pallas as pl
from jax.experimental.pallas import tpu as pltpu
from jax.experimental.pallas import tpu_sc as plsc

D = 64
L = 50
_info = plsc.get_sparse_core_info()
NC, NS = _info.num_cores, _info.num_subcores
NW = NC * NS

NB = 8
NBUF = 2


def _make_sc_call(b_total: int):
    rows_w = b_total // NW
    n_chunks = rows_w // NB
    mesh = plsc.VectorSubcoreMesh(core_axis_name="c", subcore_axis_name="s")

    @functools.partial(
        pl.kernel,
        out_type=(
            jax.ShapeDtypeStruct((b_total, L, D), jnp.float32),
            jax.ShapeDtypeStruct((b_total, L, D), jnp.float32),
        ),
        mesh=mesh,
        scratch_types=[
            pltpu.VMEM((NBUF, NB, L), jnp.int32),
            pltpu.VMEM((NBUF, NB, L), jnp.int32),
            pltpu.VMEM((NBUF, NB, L, D), jnp.float32),
            pltpu.VMEM((NBUF, NB, L, D), jnp.float32),
            pltpu.SemaphoreType.DMA((NBUF,)),
            pltpu.SemaphoreType.DMA((NBUF,)),
            pltpu.SemaphoreType.DMA((NBUF,)),
            pltpu.SemaphoreType.DMA((NBUF,)),
        ],
        compiler_params=pltpu.CompilerParams(use_tc_tiling_on_sc=False),
    )
    def sc_call(users_h, types_h, utab_h, ttab_h, out1_h, out2_h,
                idx_u, idx_t, urows, trows, gsem_u, gsem_t, osem1, osem2):
        wid = lax.axis_index("s") * NC + lax.axis_index("c")
        base_w = wid * rows_w

        def fire(ci, b):
            b0 = base_w + ci * NB
            pltpu.sync_copy(users_h.at[pl.ds(b0, NB)], idx_u.at[b])
            pltpu.sync_copy(types_h.at[pl.ds(b0, NB)], idx_t.at[b])
            for r in range(NB):
                pltpu.async_copy(utab_h.at[idx_u.at[b, r]],
                                 urows.at[b, r], gsem_u.at[b])
                pltpu.async_copy(ttab_h.at[idx_t.at[b, r]],
                                 trows.at[b, r], gsem_t.at[b])

        def wait_gathers(b):
            for r in range(NB):
                pltpu.make_async_copy(utab_h.at[idx_u.at[b, r]],
                                      urows.at[b, r], gsem_u.at[b]).wait()
                pltpu.make_async_copy(ttab_h.at[idx_t.at[b, r]],
                                      trows.at[b, r], gsem_t.at[b]).wait()

        def wait_out(ci, b):
            b0 = base_w + ci * NB
            pltpu.make_async_copy(urows.at[b], out1_h.at[pl.ds(b0, NB)],
                                  osem1.at[b]).wait()
            pltpu.make_async_copy(trows.at[b], out2_h.at[pl.ds(b0, NB)],
                                  osem2.at[b]).wait()

        fire(0, 0)

        def superstep(s, carry):
            for bb in range(NBUF):
                i = s * NBUF + bb
                b0 = base_w + i * NB
                wait_gathers(bb)

                nb_slot = (bb + 1) % NBUF

                @pl.when(i + 1 < n_chunks)
                def _(i=i, nb_slot=nb_slot):
                    @pl.when(i >= NBUF - 1)
                    def _():
                        wait_out(i + 1 - NBUF, nb_slot)
                    fire(i + 1, nb_slot)

                def add_row(j, c2, bb=bb):
                    r = j // L
                    l = j - r * L
                    for c in range(D // 16):
                        sl = pl.ds(c * 16, 16)
                        urows[bb, r, l, sl] = (urows[bb, r, l, sl]
                                               + trows[bb, r, l, sl])
                    return c2

                lax.fori_loop(0, NB * L, add_row, 0, unroll=2)
                pltpu.async_copy(urows.at[bb], out1_h.at[pl.ds(b0, NB)],
                                 osem1.at[bb])
                pltpu.async_copy(trows.at[bb], out2_h.at[pl.ds(b0, NB)],
                                 osem2.at[bb])
            return carry

        lax.fori_loop(0, n_chunks // NBUF, superstep, 0)

        for k in range(NBUF):
            ci = n_chunks - NBUF + k
            wait_out(ci, ci % NBUF)

    return sc_call


_sc_call = _make_sc_call(4096)


@jax.jit
def kernel(users, items, types, user_table, item_table, type_table):
    del items, item_table
    u = users.astype(jnp.int32)
    t = types.astype(jnp.int32)
    return _sc_call(u, t, user_table, type_table)
